# baseline (speedup 1.0000x reference)
import numpy as np
import jax
import jax.numpy as jnp
from jax import lax
from jax.experimental import pallas as pl

G = 64
K = 5
P = 10
M = 3
T = 20
TEMP = 0.1
DELTA = 1e-06
BMIN = -5.0
BMAX = 5.0

_t9 = np.arange(-4, 5, dtype=np.float32)
_gk = np.exp(-0.5 * _t9 ** 2).astype(np.float32)
_gk = (_gk / _gk.sum()).astype(np.float32)

_A = np.zeros((G + 8, G), np.float32)
for _i in range(G + 8):
    _j = _i - 4
    if _j < 0:
        _j = -_j
    if _j > G - 1:
        _j = 2 * (G - 1) - _j
    _A[_i, _j] = 1.0
_Cv = np.zeros((G, G + 8), np.float32)
for _o in range(G):
    _Cv[_o, _o:_o + 9] = _gk
_BLUR = (_Cv @ _A).astype(np.float32)

_i5 = np.arange(K, dtype=np.float32)
_FY = np.clip(G // 2 + np.trunc(10.0 * np.cos(2 * np.pi * _i5 / K)), 0, G - 1).astype(np.float32)
_FX = np.clip(G // 2 + np.trunc(10.0 * np.sin(2 * np.pi * _i5 / K)), 0, G - 1).astype(np.float32)

_pairs = [(i, j) for i in range(K) for j in range(K) if i != j][:P]
_SI = [p[0] for p in _pairs]
_SJ = [p[1] for p in _pairs]
_TLIN = np.linspace(0.0, 1.0, T).astype(np.float32).reshape(1, T)


def _mlp_kernel(zp_ref, zc_ref, w1_ref, b1_ref, w2_ref, b2_ref,
                wd_ref, bd_ref, o_ref):
    z = jnp.concatenate([zp_ref[...], zc_ref[...]], axis=-1)
    h = jnp.dot(z, w1_ref[...], preferred_element_type=jnp.float32)
    h = jnp.maximum(h + b1_ref[...], 0.0)
    h = jnp.dot(h, w2_ref[...], preferred_element_type=jnp.float32)
    h = jnp.maximum(h + b2_ref[...], 0.0)
    o = jnp.dot(h, wd_ref[...], preferred_element_type=jnp.float32)
    o_ref[...] = jnp.maximum(o + bd_ref[...], 0.0)


def _combine_v0(t):
    return ((t[0] + t[1]) + t[2]) + t[3]


def _combine_v1(t):
    return ((t[3] + t[2]) + t[1]) + t[0]


def _combine_v2(t):
    return (t[0] + t[1]) + (t[2] + t[3])


def _combine_v3(t):
    return ((t[0] + t[2]) + t[1]) + t[3]


def _combine_v4(t):
    return ((t[3] + t[1]) + t[2]) + t[0]


def _combine_v5(t):
    return (t[0] + t[2]) + (t[1] + t[3])


def _make_deconv(H, W, Cin, Cout, act, combine=_combine_v0):
    def kern(x_ref, wt_ref, b_ref, o_ref):
        x = x_ref[...]
        R = x.shape[0]
        ri = lax.broadcasted_iota(jnp.int32, (R, 1), 0)
        n = ri % W
        m = (ri // W) % H
        terms = {(py, px): [] for py in (0, 1) for px in (0, 1)}
        for dy in (-1, 0, 1):
            for dx in (-1, 0, 1):
                s = dy * W + dx
                if s == 0:
                    sx = x
                elif s > 0:
                    sx = jnp.concatenate(
                        [x[s:], jnp.zeros((s, Cin), jnp.float32)], axis=0)
                else:
                    sx = jnp.concatenate(
                        [jnp.zeros((-s, Cin), jnp.float32), x[:s]], axis=0)
                valid = None
                if dx == 1:
                    valid = n < (W - 1)
                elif dx == -1:
                    valid = n > 0
                if dy == 1:
                    vy = m < (H - 1)
                    valid = vy if valid is None else jnp.logical_and(valid, vy)
                elif dy == -1:
                    vy = m > 0
                    valid = vy if valid is None else jnp.logical_and(valid, vy)
                if valid is not None:
                    sx = jnp.where(valid, sx, 0.0)
                for py in (0, 1):
                    ky = 2 * (dy + 1) - py
                    if not 0 <= ky < 4:
                        continue
                    for px in (0, 1):
                        kx = 2 * (dx + 1) - px
                        if not 0 <= kx < 4:
                            continue
                        wk = wt_ref[ky * 4 + kx]
                        terms[(py, px)].append(
                            (ky, kx, jnp.dot(sx, wk,
                                             preferred_element_type=jnp.float32)))
        b = b_ref[...]
        outs = []
        for py in (0, 1):
            for px in (0, 1):
                ts = [t for (_, _, t) in sorted(terms[(py, px)],
                                                key=lambda e: (e[0], e[1]))]
                v = combine(ts) + b
                if act == 'relu':
                    v = jnp.maximum(v, 0.0)
                else:
                    v = jax.nn.sigmoid(v)
                outs.append(v)
        o_ref[...] = jnp.concatenate(outs, axis=1)
    return kern


def _post_kernel(f_ref, bm_ref, ty_ref, tx_ref, py_ref, px_ref, w_ref):
    f = f_ref[0]
    Bm = bm_ref[...]
    sm = jnp.dot(Bm, f, preferred_element_type=jnp.float32)
    sm = lax.dot_general(sm, Bm, (((1,), (1,)), ((), ())),
                         preferred_element_type=jnp.float32)
    pr = jnp.concatenate([sm[2:3], sm[1:2], sm, sm[62:63], sm[61:62]], axis=0)
    pc = jnp.concatenate([pr[:, 2:3], pr[:, 1:2], pr, pr[:, 62:63], pr[:, 61:62]],
                         axis=1)
    lm = pc[0:64, 0:64]
    for dy in range(5):
        for dx in range(5):
            if dy == 0 and dx == 0:
                continue
            lm = jnp.maximum(lm, pc[dy:dy + 64, dx:dx + 64])
    peaks = jnp.logical_and(sm == lm, sm > 0.3)
    vals = jnp.where(peaks, sm, -jnp.inf)
    iy = lax.broadcasted_iota(jnp.int32, (G, G), 0)
    ix = lax.broadcasted_iota(jnp.int32, (G, G), 1)
    flat = iy * G + ix
    wys, wxs = [], []
    for k in range(K):
        mv = jnp.max(vals)
        idx = jnp.min(jnp.where(vals == mv, flat, G * G))
        valid = jnp.isfinite(mv)
        yk = (idx // G).astype(jnp.float32)
        xk = (idx % G).astype(jnp.float32)
        wys.append(jnp.where(valid, yk, float(_FY[k])))
        wxs.append(jnp.where(valid, xk, float(_FX[k])))
        vals = jnp.where(flat == idx, -jnp.inf, vals)
    tl = lax.broadcasted_iota(jnp.int32, (1, T), 1).astype(jnp.float32) * (1.0 / (T - 1))
    PYs, PXs = [], []
    for p in range(P):
        sy, ey = wys[_SI[p]], wys[_SJ[p]]
        sx_, ex = wxs[_SI[p]], wxs[_SJ[p]]
        PYs.append(sy * (1.0 - tl) + ey * tl)
        PXs.append(sx_ * (1.0 - tl) + ex * tl)
    PY = jnp.concatenate(PYs, axis=0)
    PX = jnp.concatenate(PXs, axis=0)

    def smooth(a):
        inner = (a[:, :-2] + a[:, 1:-1] + a[:, 2:]) / 3.0
        return jnp.clip(jnp.concatenate([a[:, :1], inner, a[:, -1:]], axis=1),
                        0.0, float(G - 1))
    SY = smooth(PY)
    SX = smooth(PX)
    y0 = jnp.floor(SY).astype(jnp.int32)
    x0 = jnp.floor(SX).astype(jnp.int32)
    y1 = jnp.minimum(y0 + 1, G - 1)
    x1 = jnp.minimum(x0 + 1, G - 1)
    c64 = lax.broadcasted_iota(jnp.int32, (P, T, G), 2)

    def rowsel(yi):
        oh = (c64 == yi[:, :, None]).astype(jnp.float32)
        return lax.dot_general(oh, f, (((2,), (0,)), ((), ())),
                               preferred_element_type=jnp.float32)

    r0 = rowsel(y0)
    r1 = rowsel(y1)

    def colpick(rows, xi):
        oh = (c64 == xi[:, :, None]).astype(jnp.float32)
        return jnp.sum(rows * oh, axis=2)

    f00 = colpick(r0, x0)
    f01 = colpick(r0, x1)
    f10 = colpick(r1, x0)
    f11 = colpick(r1, x1)
    wy = SY - y0.astype(jnp.float32)
    wx = SX - x0.astype(jnp.float32)
    fv = (f00 * (1 - wy) * (1 - wx) + f01 * (1 - wy) * wx +
          f10 * wy * (1 - wx) + f11 * wy * wx)
    scores = jnp.sum(jnp.log(fv + DELTA), axis=1, keepdims=True) / float(T)
    dy4 = SY[:, None, :, None] - SY[None, :, None, :]
    dx4 = SX[:, None, :, None] - SX[None, :, None, :]
    d2 = dy4 * dy4 + dx4 * dx4
    dAB2 = jnp.max(jnp.min(d2, axis=3), axis=2)
    H2 = jnp.maximum(dAB2, dAB2.T)
    i10 = lax.broadcasted_iota(jnp.int32, (P, 1), 0)
    i10r = lax.broadcasted_iota(jnp.int32, (1, P), 1)
    m0 = jnp.max(scores)
    sel0 = jnp.min(jnp.where(scores == m0, i10, P))
    sels = [sel0]
    maskc = i10 == sel0
    maskr = i10r == sel0
    for _ in range(M - 1):
        mind = jnp.min(jnp.where(maskr, H2, jnp.inf), axis=1, keepdims=True)
        mind = jnp.where(maskc, -jnp.inf, mind)
        mm = jnp.max(mind)
        nxt = jnp.min(jnp.where(mind == mm, i10, P))
        sels.append(nxt)
        maskc = jnp.logical_or(maskc, i10 == nxt)
        maskr = jnp.logical_or(maskr, i10r == nxt)
    scsel = [jnp.sum(jnp.where(i10 == sk, scores, 0.0)) / TEMP for sk in sels]
    mx = jnp.maximum(jnp.maximum(scsel[0], scsel[1]), scsel[2])
    es = [jnp.exp(s - mx) for s in scsel]
    ssum = es[0] + es[1] + es[2]
    ws = [e / ssum for e in es]
    iP20 = lax.broadcasted_iota(jnp.int32, (P, T), 0)
    TY = jnp.concatenate(
        [jnp.sum(jnp.where(iP20 == sk, SY, 0.0), axis=0, keepdims=True)
         for sk in sels], axis=0)
    TX = jnp.concatenate(
        [jnp.sum(jnp.where(iP20 == sk, SX, 0.0), axis=0, keepdims=True)
         for sk in sels], axis=0)
    aw = jnp.where(ws[1] > ws[0], 1, 0)
    aw = jnp.where(ws[2] > jnp.maximum(ws[0], ws[1]), 2, aw)
    i3 = lax.broadcasted_iota(jnp.int32, (M, T), 0)
    pY = jnp.sum(jnp.where(i3 == aw, TY, 0.0), axis=0, keepdims=True)
    pX = jnp.sum(jnp.where(i3 == aw, TX, 0.0), axis=0, keepdims=True)
    scale = (BMAX - BMIN) / (G - 1)
    ty_ref[0] = BMIN + TY * scale
    tx_ref[0] = BMIN + TX * scale
    py_ref[0] = BMIN + pY * scale
    px_ref[0] = BMIN + pX * scale
    i3r = lax.broadcasted_iota(jnp.int32, (1, M), 1)
    w_ref[0] = jnp.where(i3r == 0, ws[0], jnp.where(i3r == 1, ws[1], ws[2]))


def _make_deconv_cat(H, W, Cin, Cout, act, rev=False):
    def kern(x_ref, wt_ref, b_ref, o_ref):
        x = x_ref[...]
        R = x.shape[0]
        ri = lax.broadcasted_iota(jnp.int32, (R, 1), 0)
        n = ri % W
        m = (ri // W) % H
        shifted = {}
        for dy in (-1, 0, 1):
            for dx in (-1, 0, 1):
                s = dy * W + dx
                if s == 0:
                    sx = x
                elif s > 0:
                    sx = jnp.concatenate(
                        [x[s:], jnp.zeros((s, Cin), jnp.float32)], axis=0)
                else:
                    sx = jnp.concatenate(
                        [jnp.zeros((-s, Cin), jnp.float32), x[:s]], axis=0)
                valid = None
                if dx == 1:
                    valid = n < (W - 1)
                elif dx == -1:
                    valid = n > 0
                if dy == 1:
                    vy = m < (H - 1)
                    valid = vy if valid is None else jnp.logical_and(valid, vy)
                elif dy == -1:
                    vy = m > 0
                    valid = vy if valid is None else jnp.logical_and(valid, vy)
                if valid is not None:
                    sx = jnp.where(valid, sx, 0.0)
                shifted[(dy, dx)] = sx
        b = b_ref[...]
        outs = []
        for py in (0, 1):
            for px in (0, 1):
                taps = []
                for dy in (-1, 0, 1):
                    ky = 2 * (dy + 1) - py
                    if not 0 <= ky < 4:
                        continue
                    for dx in (-1, 0, 1):
                        kx = 2 * (dx + 1) - px
                        if not 0 <= kx < 4:
                            continue
                        taps.append((ky, kx, shifted[(dy, dx)]))
                taps.sort(key=lambda e: (e[0], e[1]), reverse=rev)
                sxcat = jnp.concatenate([t[2] for t in taps], axis=1)
                wcat = jnp.concatenate([wt_ref[ky * 4 + kx]
                                        for (ky, kx, _) in taps], axis=0)
                v = jnp.dot(sxcat, wcat,
                            preferred_element_type=jnp.float32) + b
                if act == 'relu':
                    v = jnp.maximum(v, 0.0)
                else:
                    v = jax.nn.sigmoid(v)
                outs.append(v)
        o_ref[...] = jnp.concatenate(outs, axis=1)
    return kern


def _deconv_call(x2d, wt, b2d, H, W, Cin, Cout, act, nb, combine=_combine_v0,
                 cat=False, rev=False):
    R = x2d.shape[0]
    Rblk = R // nb
    return pl.pallas_call(
        _make_deconv_cat(H, W, Cin, Cout, act, rev) if cat
        else _make_deconv(H, W, Cin, Cout, act, combine),
        grid=(nb,),
        in_specs=[
            pl.BlockSpec((Rblk, Cin), lambda i: (i, 0)),
            pl.BlockSpec((16, Cin, Cout), lambda i: (0, 0, 0)),
            pl.BlockSpec((1, Cout), lambda i: (0, 0)),
        ],
        out_specs=pl.BlockSpec((Rblk, 4 * Cout), lambda i: (i, 0)),
        out_shape=jax.ShapeDtypeStruct((R, 4 * Cout), jnp.float32),
    )(x2d, wt, b2d)


def _interleave(o2d, B, H, W, Cout):
    o = o2d.reshape(B, H, W, 2, 2, Cout).transpose(0, 1, 3, 2, 4, 5)
    return o.reshape(B, 2 * H, 2 * W, Cout)


def kernel(z_pooled, z_c, W1, b1, W2, b2, Wd, bd,
           C1, cb1, C2, cb2, C3, cb3, C4, cb4):
    B = z_pooled.shape[0]
    x4096 = pl.pallas_call(
        _mlp_kernel,
        out_shape=jax.ShapeDtypeStruct((B, 4096), jnp.float32),
    )(z_pooled, z_c, W1, b1.reshape(1, -1),
      W2, b2.reshape(1, -1), Wd, bd.reshape(1, -1))
    x = x4096.reshape(B, 256, 4, 4).transpose(0, 2, 3, 1).reshape(B * 16, 256)

    layers = [
        (C1, cb1, 4, 4, 256, 128, 'relu', 1),
        (C2, cb2, 8, 8, 128, 64, 'relu', 4),
        (C3, cb3, 16, 16, 64, 32, 'relu', 8),
        (C4, cb4, 32, 32, 32, 1, 'sigmoid', 16),
    ]
    for (Cw, cb, H, W, Cin, Cout, act, nb) in layers:
        wt = jnp.transpose(Cw, (2, 3, 1, 0)).reshape(16, Cin, Cout)
        o2d = _deconv_call(x, wt, cb.reshape(1, -1), H, W, Cin, Cout, act, nb)
        o = _interleave(o2d, B, H, W, Cout)
        x = o.reshape(B * 2 * H * 2 * W, Cout)
    field = o[..., 0]

    tY, tX, pY, pX, wv = pl.pallas_call(
        _post_kernel,
        grid=(B,),
        in_specs=[
            pl.BlockSpec((1, G, G), lambda b: (b, 0, 0)),
            pl.BlockSpec((G, G), lambda b: (0, 0)),
        ],
        out_specs=[
            pl.BlockSpec((1, M, T), lambda b: (b, 0, 0)),
            pl.BlockSpec((1, M, T), lambda b: (b, 0, 0)),
            pl.BlockSpec((1, 1, T), lambda b: (b, 0, 0)),
            pl.BlockSpec((1, 1, T), lambda b: (b, 0, 0)),
            pl.BlockSpec((1, 1, M), lambda b: (b, 0, 0)),
        ],
        out_shape=[
            jax.ShapeDtypeStruct((B, M, T), jnp.float32),
            jax.ShapeDtypeStruct((B, M, T), jnp.float32),
            jax.ShapeDtypeStruct((B, 1, T), jnp.float32),
            jax.ShapeDtypeStruct((B, 1, T), jnp.float32),
            jax.ShapeDtypeStruct((B, 1, M), jnp.float32),
        ],
    )(field, jnp.asarray(_BLUR))

    trajs = jnp.stack([tY, tX], axis=-1)
    prior = jnp.stack([pY[:, 0], pX[:, 0]], axis=-1)
    w = wv[:, 0]
    return (prior, field, trajs, w)
